# Initial kernel scaffold; baseline (speedup 1.0000x reference)
import jax
import jax.numpy as jnp
from jax.experimental import pallas as pl


def kernel(x, edge_index, W1, b1, W2, b2):
    raise NotImplementedError("write your pallas kernel here")


# speedup vs baseline: 20.5213x; 20.5213x over previous
import functools

import jax
import jax.numpy as jnp
from jax import lax
from jax.experimental import pallas as pl
from jax.experimental.pallas import tpu as pltpu
from jax.experimental.pallas import tpu_sc as plsc

N = 10000
E = 320000
D_IN = 128
HIDDEN = 64

NC = 2
NS = 16
NW = NC * NS
L = 16

F_PER = HIDDEN // NW
E_PER_W = E // NW
E_PER_C = E // NC
CH = 2000

_SC_PARAMS = pltpu.CompilerParams(needs_layout_passes=False)

_mesh = functools.partial(
    plsc.VectorSubcoreMesh, core_axis_name="c", subcore_axis_name="s",
    num_cores=NC, num_subcores=NS,
)


def _zero_1d(ref, n):
    z = jnp.zeros((L,), jnp.float32)

    def body(i, _):
        ref[pl.ds(i * L, L)] = z
        return 0

    lax.fori_loop(0, n // L, body, 0, unroll=4)


def _deg_body(dst_hbm, out_hbm, dst_v, acc_v):
    c = lax.axis_index("c")
    s = lax.axis_index("s")
    wid = c * NS + s
    base = wid * E_PER_W
    _zero_1d(acc_v, N)
    ones = jnp.ones((L,), jnp.float32)

    def chunk(k, _):
        pltpu.sync_copy(dst_hbm.at[pl.ds(base + k * CH, CH)], dst_v)

        def step(i, _):
            d16 = dst_v[pl.ds(i * L, L)]
            plsc.addupdate_scatter(acc_v, [d16], ones)
            return 0

        lax.fori_loop(0, CH // L, step, 0, unroll=4)
        return 0

    lax.fori_loop(0, E_PER_W // CH, chunk, 0)
    pltpu.sync_copy(acc_v, out_hbm.at[c, s])


_deg_call = pl.kernel(
    _deg_body,
    out_type=jax.ShapeDtypeStruct((NC, NS, N), jnp.float32),
    mesh=_mesh(),
    scratch_types=[
        pltpu.VMEM((CH,), jnp.int32),
        pltpu.VMEM((N,), jnp.float32),
    ],
    compiler_params=_SC_PARAMS,
)


FBLK = F_PER * NC


def _agg1_body(hT_hbm, src_hbm, dst_hbm, dinv_hbm, out_hbm,
               h_v, dinv_v, src_v, dst_v, acc_v):
    c = lax.axis_index("c")
    s = lax.axis_index("s")
    base = c * E_PER_C
    pltpu.sync_copy(hT_hbm.at[s], h_v)
    pltpu.sync_copy(dinv_hbm, dinv_v)
    z = jnp.zeros((L,), jnp.float32)

    def zbody(i, _):
        for f in range(FBLK):
            acc_v[f, pl.ds(i * L, L)] = z
        return 0

    lax.fori_loop(0, N // L, zbody, 0, unroll=2)

    fidx = [jnp.full((L,), f, jnp.int32) for f in range(FBLK)]

    def chunk(k, _):
        pltpu.sync_copy(src_hbm.at[pl.ds(base + k * CH, CH)], src_v)
        pltpu.sync_copy(dst_hbm.at[pl.ds(base + k * CH, CH)], dst_v)

        def step(i, _):
            s16 = src_v[pl.ds(i * L, L)]
            d16 = dst_v[pl.ds(i * L, L)]
            nrm = plsc.load_gather(dinv_v, [s16]) * plsc.load_gather(dinv_v, [d16])
            for f in range(FBLK):
                g = plsc.load_gather(h_v, [fidx[f], s16])
                plsc.addupdate_scatter(acc_v, [fidx[f], d16], g * nrm)
            return 0

        lax.fori_loop(0, CH // L, step, 0, unroll=2)
        return 0

    lax.fori_loop(0, E_PER_C // CH, chunk, 0)
    pltpu.sync_copy(acc_v, out_hbm.at[c, s])


_agg1_call = pl.kernel(
    _agg1_body,
    out_type=jax.ShapeDtypeStruct((NC, NS, FBLK, N), jnp.float32),
    mesh=_mesh(),
    scratch_types=[
        pltpu.VMEM((FBLK, N), jnp.float32),
        pltpu.VMEM((N,), jnp.float32),
        pltpu.VMEM((CH,), jnp.int32),
        pltpu.VMEM((CH,), jnp.int32),
        pltpu.VMEM((FBLK, N), jnp.float32),
    ],
    compiler_params=_SC_PARAMS,
)


def _agg2_body(s_hbm, src_hbm, dst_hbm, dinv_hbm, out_hbm,
               sv_v, dinv_v, src_v, dst_v, acc_v):
    c = lax.axis_index("c")
    s = lax.axis_index("s")
    wid = c * NS + s
    base = wid * E_PER_W
    pltpu.sync_copy(s_hbm, sv_v)
    pltpu.sync_copy(dinv_hbm, dinv_v)
    _zero_1d(acc_v, N)

    def chunk(k, _):
        pltpu.sync_copy(src_hbm.at[pl.ds(base + k * CH, CH)], src_v)
        pltpu.sync_copy(dst_hbm.at[pl.ds(base + k * CH, CH)], dst_v)

        def step(i, _):
            s16 = src_v[pl.ds(i * L, L)]
            d16 = dst_v[pl.ds(i * L, L)]
            nrm = plsc.load_gather(dinv_v, [s16]) * plsc.load_gather(dinv_v, [d16])
            val = plsc.load_gather(sv_v, [s16])
            plsc.addupdate_scatter(acc_v, [d16], val * nrm)
            return 0

        lax.fori_loop(0, CH // L, step, 0, unroll=4)
        return 0

    lax.fori_loop(0, E_PER_W // CH, chunk, 0)
    pltpu.sync_copy(acc_v, out_hbm.at[c, s])


_agg2_call = pl.kernel(
    _agg2_body,
    out_type=jax.ShapeDtypeStruct((NC, NS, N), jnp.float32),
    mesh=_mesh(),
    scratch_types=[
        pltpu.VMEM((N,), jnp.float32),
        pltpu.VMEM((N,), jnp.float32),
        pltpu.VMEM((CH,), jnp.int32),
        pltpu.VMEM((CH,), jnp.int32),
        pltpu.VMEM((N,), jnp.float32),
    ],
    compiler_params=_SC_PARAMS,
)


def _mm_body(w_ref, x_ref, o_ref):
    o_ref[...] = lax.dot_general(
        w_ref[...], x_ref[...], (((1,), (1,)), ((), ())),
        preferred_element_type=jnp.float32,
    )


def _dinv_body(degp_ref, dinv_ref, dinv2_ref):
    d = jnp.sum(degp_ref[...], axis=0, keepdims=True) + 1.0
    dinv_ref[...] = lax.rsqrt(d)
    dinv2_ref[...] = 1.0 / d


def _mid_body(agg_ref, h_ref, dinv2_ref, b1_ref, w2_ref, s_ref):
    a = agg_ref[0:HIDDEN, :] + agg_ref[HIDDEN:2 * HIDDEN, :]
    r = a + dinv2_ref[...] * h_ref[...] + b1_ref[...]
    r = jnp.maximum(r, 0.0)
    s_ref[...] = lax.dot_general(
        w2_ref[...], r, (((1,), (0,)), ((), ())),
        preferred_element_type=jnp.float32,
    )


def _final_body(aggp_ref, s_ref, dinv2_ref, b2_ref, o_ref):
    o_ref[...] = (jnp.sum(aggp_ref[...], axis=0, keepdims=True)
                  + dinv2_ref[...] * s_ref[...] + b2_ref[...])


def kernel(x, edge_index, W1, b1, W2, b2):
    src = edge_index[0].astype(jnp.int32)
    dst = edge_index[1].astype(jnp.int32)

    degp = _deg_call(dst)

    hT = pl.pallas_call(
        _mm_body,
        out_shape=jax.ShapeDtypeStruct((HIDDEN, N), jnp.float32),
    )(W1, x)

    dinv2d, dinv2_2d = pl.pallas_call(
        _dinv_body,
        out_shape=[
            jax.ShapeDtypeStruct((1, N), jnp.float32),
            jax.ShapeDtypeStruct((1, N), jnp.float32),
        ],
    )(degp.reshape(NW, N))
    dinv = dinv2d.reshape(N)

    agg1 = _agg1_call(hT.reshape(NS, FBLK, N), src, dst, dinv)

    s2d = pl.pallas_call(
        _mid_body,
        out_shape=jax.ShapeDtypeStruct((1, N), jnp.float32),
    )(agg1.reshape(NC * HIDDEN, N), hT, dinv2_2d, b1.reshape(HIDDEN, 1), W2)

    agg2 = _agg2_call(s2d.reshape(N), src, dst, dinv)

    out2d = pl.pallas_call(
        _final_body,
        out_shape=jax.ShapeDtypeStruct((1, N), jnp.float32),
    )(agg2.reshape(NW, N), s2d, dinv2_2d, b2.reshape(1, 1))
    return out2d.reshape(N, 1)


# speedup vs baseline: 65.5534x; 3.1944x over previous
import functools

import jax
import jax.numpy as jnp
from jax import lax
from jax.experimental import pallas as pl
from jax.experimental.pallas import tpu as pltpu
from jax.experimental.pallas import tpu_sc as plsc

N = 10000
E = 320000
D_IN = 128
HIDDEN = 64

NC = 2
NS = 16
NW = NC * NS
L = 16

E_PER_W = E // NW
BATCH = 100
NBATCH = E_PER_W // BATCH
CH2 = 2000
RB = 1000
NRB = N // RB

_SC_PARAMS = pltpu.CompilerParams(needs_layout_passes=False)
_SC_PARAMS_NT = pltpu.CompilerParams(
    needs_layout_passes=False, use_tc_tiling_on_sc=False
)

_mesh = functools.partial(
    plsc.VectorSubcoreMesh, core_axis_name="c", subcore_axis_name="s",
    num_cores=NC, num_subcores=NS,
)


def _zero_1d(ref, n):
    z = jnp.zeros((L,), jnp.float32)

    def body(i, _):
        ref[pl.ds(i * L, L)] = z
        return 0

    lax.fori_loop(0, n // L, body, 0, unroll=8)


def _deg_body(dst_hbm, out_hbm, dst_v0, dst_v1, acc_v, sems):
    c = lax.axis_index("c")
    s = lax.axis_index("s")
    base = (c * NS + s) * E_PER_W
    nchunks = E_PER_W // CH2
    bufs = (dst_v0, dst_v1)

    for b in range(2):
        pltpu.async_copy(
            dst_hbm.at[pl.ds(base + b * CH2, CH2)], bufs[b], sems.at[b]
        )
    _zero_1d(acc_v, N)
    ones = jnp.ones((L,), jnp.float32)

    for k in range(nchunks):
        b = k % 2
        pltpu.make_async_copy(
            dst_hbm.at[pl.ds(base + k * CH2, CH2)], bufs[b], sems.at[b]
        ).wait()

        def step(i, _, b=b):
            d16 = bufs[b][pl.ds(i * L, L)]
            plsc.addupdate_scatter(acc_v, [d16], ones)
            return 0

        lax.fori_loop(0, CH2 // L, step, 0, unroll=8)
        if k + 2 < nchunks:
            pltpu.async_copy(
                dst_hbm.at[pl.ds(base + (k + 2) * CH2, CH2)],
                bufs[b], sems.at[b],
            )
    pltpu.sync_copy(acc_v, out_hbm.at[c, s])


_deg_call = pl.kernel(
    _deg_body,
    out_type=jax.ShapeDtypeStruct((NC, NS, N), jnp.float32),
    mesh=_mesh(),
    scratch_types=[
        pltpu.VMEM((CH2,), jnp.int32),
        pltpu.VMEM((CH2,), jnp.int32),
        pltpu.VMEM((N,), jnp.float32),
        pltpu.SemaphoreType.DMA((2,)),
    ],
    compiler_params=_SC_PARAMS,
)


def _agg1_body(hp_hbm, src3d_hbm, dst3d_hbm, out_hbm,
               sidx, didx, msg0, msg1, acc_sh, sem0, sem1, gsems):
    c = lax.axis_index("c")
    s = lax.axis_index("s")
    wid = c * NS + s

    idma = pltpu.async_copy(src3d_hbm.at[wid], sidx, sem0)
    jdma = pltpu.async_copy(dst3d_hbm.at[wid], didx, sem1)

    @pl.when(s < NRB)
    def _prologue():
        row0 = s * RB
        pltpu.sync_copy(hp_hbm.at[pl.ds(row0, RB)], acc_sh.at[pl.ds(row0, RB)])

    idma.wait()
    jdma.wait()
    plsc.subcore_barrier()

    msgs = (msg0, msg1)
    for b in range(2):
        pltpu.async_copy(hp_hbm.at[sidx.at[b]], msgs[b], gsems.at[b])

    @pl.loop(0, NBATCH, step=2)
    def _batches(j0):
        for b in range(2):
            j = j0 + b
            pltpu.make_async_copy(
                hp_hbm.at[sidx.at[j]], msgs[b], gsems.at[b]
            ).wait()
            pltpu.sync_copy(msgs[b], acc_sh.at[didx.at[j]], add=True)

            @pl.when(j + 2 < NBATCH)
            def _():
                pltpu.async_copy(
                    hp_hbm.at[sidx.at[j + 2]], msgs[b], gsems.at[b]
                )

    plsc.subcore_barrier()

    @pl.when(s < NRB)
    def _epilogue():
        row0 = s * RB
        pltpu.sync_copy(acc_sh.at[pl.ds(row0, RB)], out_hbm.at[c, s])


_agg1_call = pl.kernel(
    _agg1_body,
    out_type=jax.ShapeDtypeStruct((NC, NRB, RB, HIDDEN), jnp.float32),
    mesh=_mesh(),
    scratch_types=[
        pltpu.VMEM((NBATCH, BATCH), jnp.int32),
        pltpu.VMEM((NBATCH, BATCH), jnp.int32),
        pltpu.VMEM((BATCH, HIDDEN), jnp.float32),
        pltpu.VMEM((BATCH, HIDDEN), jnp.float32),
        pltpu.VMEM_SHARED((N, HIDDEN), jnp.float32),
        pltpu.SemaphoreType.DMA,
        pltpu.SemaphoreType.DMA,
        pltpu.SemaphoreType.DMA((2,)),
    ],
    compiler_params=_SC_PARAMS_NT,
)


def _agg2_body(sp_hbm, src_hbm, dst_hbm, out_hbm, sv_v,
               sbuf0, sbuf1, dbuf0, dbuf1, acc_v, hsem, sems):
    c = lax.axis_index("c")
    s = lax.axis_index("s")
    base = (c * NS + s) * E_PER_W
    nchunks = E_PER_W // CH2
    sbufs = (sbuf0, sbuf1)
    dbufs = (dbuf0, dbuf1)

    sdma = pltpu.async_copy(sp_hbm, sv_v, hsem)
    for b in range(2):
        pltpu.async_copy(
            src_hbm.at[pl.ds(base + b * CH2, CH2)], sbufs[b], sems.at[b]
        )
        pltpu.async_copy(
            dst_hbm.at[pl.ds(base + b * CH2, CH2)], dbufs[b], sems.at[b]
        )
    _zero_1d(acc_v, N)
    sdma.wait()

    for k in range(nchunks):
        b = k % 2
        pltpu.make_async_copy(
            src_hbm.at[pl.ds(base + k * CH2, CH2)], sbufs[b], sems.at[b]
        ).wait()
        pltpu.make_async_copy(
            dst_hbm.at[pl.ds(base + k * CH2, CH2)], dbufs[b], sems.at[b]
        ).wait()

        def step(i, _, b=b):
            s16 = sbufs[b][pl.ds(i * L, L)]
            d16 = dbufs[b][pl.ds(i * L, L)]
            val = plsc.load_gather(sv_v, [s16])
            plsc.addupdate_scatter(acc_v, [d16], val)
            return 0

        lax.fori_loop(0, CH2 // L, step, 0, unroll=8)
        if k + 2 < nchunks:
            pltpu.async_copy(
                src_hbm.at[pl.ds(base + (k + 2) * CH2, CH2)],
                sbufs[b], sems.at[b],
            )
            pltpu.async_copy(
                dst_hbm.at[pl.ds(base + (k + 2) * CH2, CH2)],
                dbufs[b], sems.at[b],
            )
    pltpu.sync_copy(acc_v, out_hbm.at[c, s])


_agg2_call = pl.kernel(
    _agg2_body,
    out_type=jax.ShapeDtypeStruct((NC, NS, N), jnp.float32),
    mesh=_mesh(),
    scratch_types=[
        pltpu.VMEM((N,), jnp.float32),
        pltpu.VMEM((CH2,), jnp.int32),
        pltpu.VMEM((CH2,), jnp.int32),
        pltpu.VMEM((CH2,), jnp.int32),
        pltpu.VMEM((CH2,), jnp.int32),
        pltpu.VMEM((N,), jnp.float32),
        pltpu.SemaphoreType.DMA,
        pltpu.SemaphoreType.DMA((2,)),
    ],
    compiler_params=_SC_PARAMS,
)


def _mm_body(x_ref, w_ref, o_ref):
    o_ref[...] = lax.dot_general(
        x_ref[...], w_ref[...], (((1,), (1,)), ((), ())),
        preferred_element_type=jnp.float32,
    )


def _dinv_body(degp_ref, h_ref, dinv_ref, dinv2_ref, hp_ref):
    d = jnp.sum(degp_ref[...], axis=0, keepdims=True) + 1.0
    dinv = lax.rsqrt(d)
    dinv_ref[...] = dinv
    dinv2_ref[...] = 1.0 / d
    hp_ref[...] = jnp.reshape(dinv, (N, 1)) * h_ref[...]


def _mid_body(p_ref, hp_ref, b1_ref, w2_ref, dinv_ref, s_ref, sp_ref):
    dcol = jnp.reshape(dinv_ref[...], (N, 1))
    r = dcol * (p_ref[0:N, :] + p_ref[N:2 * N, :] - hp_ref[...]) + b1_ref[...]
    r = jnp.maximum(r, 0.0)
    sv = lax.dot_general(
        w2_ref[...], r, (((1,), (1,)), ((), ())),
        preferred_element_type=jnp.float32,
    )
    s_ref[...] = sv
    sp_ref[...] = dinv_ref[...] * sv


def _final_body(aggp_ref, s_ref, dinv_ref, dinv2_ref, b2_ref, o_ref):
    o_ref[...] = (dinv_ref[...] * jnp.sum(aggp_ref[...], axis=0, keepdims=True)
                  + dinv2_ref[...] * s_ref[...] + b2_ref[...])


def kernel(x, edge_index, W1, b1, W2, b2):
    edges = edge_index.astype(jnp.int32)
    src = edges[0]
    dst = edges[1]

    degp = _deg_call(dst)

    h = pl.pallas_call(
        _mm_body,
        out_shape=jax.ShapeDtypeStruct((N, HIDDEN), jnp.float32),
    )(x, W1)

    dinv2d, dinv2_2d, hp = pl.pallas_call(
        _dinv_body,
        out_shape=[
            jax.ShapeDtypeStruct((1, N), jnp.float32),
            jax.ShapeDtypeStruct((1, N), jnp.float32),
            jax.ShapeDtypeStruct((N, HIDDEN), jnp.float32),
        ],
    )(degp.reshape(NW, N), h)

    agg1 = _agg1_call(
        hp,
        src.reshape(NW, NBATCH, BATCH), dst.reshape(NW, NBATCH, BATCH),
    )

    s2d, sp2d = pl.pallas_call(
        _mid_body,
        out_shape=[
            jax.ShapeDtypeStruct((1, N), jnp.float32),
            jax.ShapeDtypeStruct((1, N), jnp.float32),
        ],
    )(agg1.reshape(NC * N, HIDDEN), hp, b1.reshape(1, HIDDEN), W2, dinv2d)

    agg2 = _agg2_call(sp2d.reshape(N), src, dst)

    out2d = pl.pallas_call(
        _final_body,
        out_shape=jax.ShapeDtypeStruct((1, N), jnp.float32),
    )(agg2.reshape(NW, N), s2d, dinv2d, dinv2_2d, b2.reshape(1, 1))
    return out2d.reshape(N, 1)


# speedup vs baseline: 76.2292x; 1.1629x over previous
import functools

import jax
import jax.numpy as jnp
from jax import lax
from jax.experimental import pallas as pl
from jax.experimental.pallas import tpu as pltpu
from jax.experimental.pallas import tpu_sc as plsc

N = 10000
E = 320000
D_IN = 128
HIDDEN = 64

NC = 2
NS = 16
NW = NC * NS
L = 16

E_PER_W = E // NW
BATCH = 100
NBATCH = E_PER_W // BATCH
CH2 = 2000
RB = 1000
NRB = N // RB

_SC_PARAMS = pltpu.CompilerParams(needs_layout_passes=False)
_SC_PARAMS_NT = pltpu.CompilerParams(
    needs_layout_passes=False, use_tc_tiling_on_sc=False
)

_mesh = functools.partial(
    plsc.VectorSubcoreMesh, core_axis_name="c", subcore_axis_name="s",
    num_cores=NC, num_subcores=NS,
)


def _zero_1d(ref, n):
    z = jnp.zeros((L,), jnp.float32)

    def body(i, _):
        ref[pl.ds(i * L, L)] = z
        return 0

    lax.fori_loop(0, n // L, body, 0, unroll=8)


def _deg_body(dst_hbm, out_hbm, dst_v0, dst_v1, acc_v, sems):
    c = lax.axis_index("c")
    s = lax.axis_index("s")
    base = (c * NS + s) * E_PER_W
    nchunks = E_PER_W // CH2
    bufs = (dst_v0, dst_v1)

    for b in range(2):
        pltpu.async_copy(
            dst_hbm.at[pl.ds(base + b * CH2, CH2)], bufs[b], sems.at[b]
        )
    _zero_1d(acc_v, N)
    ones = jnp.ones((L,), jnp.float32)

    for k in range(nchunks):
        b = k % 2
        pltpu.make_async_copy(
            dst_hbm.at[pl.ds(base + k * CH2, CH2)], bufs[b], sems.at[b]
        ).wait()

        def step(i, _, b=b):
            d16 = bufs[b][pl.ds(i * L, L)]
            plsc.addupdate_scatter(acc_v, [d16], ones)
            return 0

        lax.fori_loop(0, CH2 // L, step, 0, unroll=8)
        if k + 2 < nchunks:
            pltpu.async_copy(
                dst_hbm.at[pl.ds(base + (k + 2) * CH2, CH2)],
                bufs[b], sems.at[b],
            )
    pltpu.sync_copy(acc_v, out_hbm.at[c, s])


_deg_call = pl.kernel(
    _deg_body,
    out_type=jax.ShapeDtypeStruct((NC, NS, N), jnp.float32),
    mesh=_mesh(),
    scratch_types=[
        pltpu.VMEM((CH2,), jnp.int32),
        pltpu.VMEM((CH2,), jnp.int32),
        pltpu.VMEM((N,), jnp.float32),
        pltpu.SemaphoreType.DMA((2,)),
    ],
    compiler_params=_SC_PARAMS,
)


def _agg1_body(hp_hbm, src3d_hbm, dst3d_hbm, out_hbm,
               sidx, didx, msg0, msg1, msg2, msg3, acc_sh,
               sem0, sem1, gsems, ssems):
    c = lax.axis_index("c")
    s = lax.axis_index("s")
    wid = c * NS + s

    idma = pltpu.async_copy(src3d_hbm.at[wid], sidx, sem0)
    jdma = pltpu.async_copy(dst3d_hbm.at[wid], didx, sem1)

    @pl.when(s < NRB)
    def _prologue():
        row0 = s * RB
        pltpu.sync_copy(hp_hbm.at[pl.ds(row0, RB)], acc_sh.at[pl.ds(row0, RB)])

    idma.wait()
    jdma.wait()
    plsc.subcore_barrier()

    msgs = (msg0, msg1, msg2, msg3)

    def issue_gather(j, b):
        pltpu.async_copy(hp_hbm.at[sidx.at[j]], msgs[b], gsems.at[b])

    def wait_gather(j, b):
        pltpu.make_async_copy(hp_hbm.at[sidx.at[j]], msgs[b], gsems.at[b]).wait()

    def issue_scatter(j, b):
        pltpu.async_copy(msgs[b], acc_sh.at[didx.at[j]], ssems.at[b], add=True)

    def wait_scatter(j, b):
        pltpu.make_async_copy(msgs[b], acc_sh.at[didx.at[j]], ssems.at[b]).wait()

    issue_gather(0, 0)
    issue_gather(1, 1)

    @pl.loop(0, NBATCH, step=4)
    def _batches(j0):
        for b in range(4):
            j = j0 + b
            bn = (b + 2) % 4

            @pl.when(j >= 2)
            def _():
                wait_scatter(j - 2, bn)

            @pl.when(j + 2 < NBATCH)
            def _():
                issue_gather(j + 2, bn)

            wait_gather(j, b)
            issue_scatter(j, b)

    wait_scatter(NBATCH - 2, (NBATCH - 2) % 4)
    wait_scatter(NBATCH - 1, (NBATCH - 1) % 4)
    plsc.subcore_barrier()

    @pl.when(s < NRB)
    def _epilogue():
        row0 = s * RB
        pltpu.sync_copy(acc_sh.at[pl.ds(row0, RB)], out_hbm.at[c, s])


_agg1_call = pl.kernel(
    _agg1_body,
    out_type=jax.ShapeDtypeStruct((NC, NRB, RB, HIDDEN), jnp.float32),
    mesh=_mesh(),
    scratch_types=[
        pltpu.VMEM((NBATCH, BATCH), jnp.int32),
        pltpu.VMEM((NBATCH, BATCH), jnp.int32),
        pltpu.VMEM((BATCH, HIDDEN), jnp.float32),
        pltpu.VMEM((BATCH, HIDDEN), jnp.float32),
        pltpu.VMEM((BATCH, HIDDEN), jnp.float32),
        pltpu.VMEM((BATCH, HIDDEN), jnp.float32),
        pltpu.VMEM_SHARED((N, HIDDEN), jnp.float32),
        pltpu.SemaphoreType.DMA,
        pltpu.SemaphoreType.DMA,
        pltpu.SemaphoreType.DMA((4,)),
        pltpu.SemaphoreType.DMA((4,)),
    ],
    compiler_params=_SC_PARAMS_NT,
)


def _agg2_body(sp_hbm, src_hbm, dst_hbm, out_hbm, sv_v,
               sbuf0, sbuf1, dbuf0, dbuf1, acc_v, hsem, sems):
    c = lax.axis_index("c")
    s = lax.axis_index("s")
    base = (c * NS + s) * E_PER_W
    nchunks = E_PER_W // CH2
    sbufs = (sbuf0, sbuf1)
    dbufs = (dbuf0, dbuf1)

    sdma = pltpu.async_copy(sp_hbm, sv_v, hsem)
    for b in range(2):
        pltpu.async_copy(
            src_hbm.at[pl.ds(base + b * CH2, CH2)], sbufs[b], sems.at[b]
        )
        pltpu.async_copy(
            dst_hbm.at[pl.ds(base + b * CH2, CH2)], dbufs[b], sems.at[b]
        )
    _zero_1d(acc_v, N)
    sdma.wait()

    for k in range(nchunks):
        b = k % 2
        pltpu.make_async_copy(
            src_hbm.at[pl.ds(base + k * CH2, CH2)], sbufs[b], sems.at[b]
        ).wait()
        pltpu.make_async_copy(
            dst_hbm.at[pl.ds(base + k * CH2, CH2)], dbufs[b], sems.at[b]
        ).wait()

        def step(i, _, b=b):
            s16 = sbufs[b][pl.ds(i * L, L)]
            d16 = dbufs[b][pl.ds(i * L, L)]
            val = plsc.load_gather(sv_v, [s16])
            plsc.addupdate_scatter(acc_v, [d16], val)
            return 0

        lax.fori_loop(0, CH2 // L, step, 0, unroll=8)
        if k + 2 < nchunks:
            pltpu.async_copy(
                src_hbm.at[pl.ds(base + (k + 2) * CH2, CH2)],
                sbufs[b], sems.at[b],
            )
            pltpu.async_copy(
                dst_hbm.at[pl.ds(base + (k + 2) * CH2, CH2)],
                dbufs[b], sems.at[b],
            )
    pltpu.sync_copy(acc_v, out_hbm.at[c, s])


_agg2_call = pl.kernel(
    _agg2_body,
    out_type=jax.ShapeDtypeStruct((NC, NS, N), jnp.float32),
    mesh=_mesh(),
    scratch_types=[
        pltpu.VMEM((N,), jnp.float32),
        pltpu.VMEM((CH2,), jnp.int32),
        pltpu.VMEM((CH2,), jnp.int32),
        pltpu.VMEM((CH2,), jnp.int32),
        pltpu.VMEM((CH2,), jnp.int32),
        pltpu.VMEM((N,), jnp.float32),
        pltpu.SemaphoreType.DMA,
        pltpu.SemaphoreType.DMA((2,)),
    ],
    compiler_params=_SC_PARAMS,
)


def _ab_body(x_ref, w_ref, degp_ref, dinv_ref, dinv2_ref, hp_ref):
    h = lax.dot_general(
        x_ref[...], w_ref[...], (((1,), (1,)), ((), ())),
        preferred_element_type=jnp.float32,
    )
    d = jnp.sum(degp_ref[...], axis=0, keepdims=True) + 1.0
    dinv = lax.rsqrt(d)
    dinv_ref[...] = dinv
    dinv2_ref[...] = 1.0 / d
    hp_ref[...] = jnp.reshape(dinv, (N, 1)) * h


def _mid_body(p_ref, hp_ref, b1_ref, w2_ref, dinv_ref, s_ref, sp_ref):
    dcol = jnp.reshape(dinv_ref[...], (N, 1))
    r = dcol * (p_ref[0:N, :] + p_ref[N:2 * N, :] - hp_ref[...]) + b1_ref[...]
    r = jnp.maximum(r, 0.0)
    sv = lax.dot_general(
        w2_ref[...], r, (((1,), (1,)), ((), ())),
        preferred_element_type=jnp.float32,
    )
    s_ref[...] = sv
    sp_ref[...] = dinv_ref[...] * sv


def _final_body(aggp_ref, s_ref, dinv_ref, dinv2_ref, b2_ref, o_ref):
    o_ref[...] = (dinv_ref[...] * jnp.sum(aggp_ref[...], axis=0, keepdims=True)
                  + dinv2_ref[...] * s_ref[...] + b2_ref[...])


def kernel(x, edge_index, W1, b1, W2, b2):
    edges = edge_index.astype(jnp.int32)
    src = edges[0]
    dst = edges[1]

    degp = _deg_call(dst)

    dinv2d, dinv2_2d, hp = pl.pallas_call(
        _ab_body,
        out_shape=[
            jax.ShapeDtypeStruct((1, N), jnp.float32),
            jax.ShapeDtypeStruct((1, N), jnp.float32),
            jax.ShapeDtypeStruct((N, HIDDEN), jnp.float32),
        ],
    )(x, W1, degp.reshape(NW, N))

    agg1 = _agg1_call(
        hp,
        src.reshape(NW, NBATCH, BATCH), dst.reshape(NW, NBATCH, BATCH),
    )

    s2d, sp2d = pl.pallas_call(
        _mid_body,
        out_shape=[
            jax.ShapeDtypeStruct((1, N), jnp.float32),
            jax.ShapeDtypeStruct((1, N), jnp.float32),
        ],
    )(agg1.reshape(NC * N, HIDDEN), hp, b1.reshape(1, HIDDEN), W2, dinv2d)

    agg2 = _agg2_call(sp2d.reshape(N), src, dst)

    out2d = pl.pallas_call(
        _final_body,
        out_shape=jax.ShapeDtypeStruct((1, N), jnp.float32),
    )(agg2.reshape(NW, N), s2d, dinv2d, dinv2_2d, b2.reshape(1, 1))
    return out2d.reshape(N, 1)
